# Initial kernel scaffold; baseline (speedup 1.0000x reference)
import jax
import jax.numpy as jnp
from jax.experimental import pallas as pl


def kernel(scores, y):
    raise NotImplementedError("write your pallas kernel here")


# speedup vs baseline: 1.2626x; 1.2626x over previous
import functools

import jax
import jax.numpy as jnp
from jax import lax
from jax.experimental import pallas as pl
from jax.experimental.pallas import tpu as pltpu
from jax.experimental.pallas import tpu_sc as plsc

B, C, H, W = 8, 96, 224, 224
HWN = H * W
NPIX = B * HWN
NC, NS, L = 2, 16, 16
NW = NC * NS
PPW = NPIX // NW
CH = 128
NCH = PPW // CH
NV = PPW // L


def _body(scores_hbm, y_hbm, out_hbm, y_v, idx_v, val_v, acc_v, sem):
    c = lax.axis_index("c")
    s = lax.axis_index("s")
    wid = s * NC + c
    base = wid * PPW
    b = base // HWN
    lin = b * (C * HWN) + (base - b * HWN)

    pltpu.sync_copy(y_hbm.at[pl.ds(base, PPW)], y_v)

    lane = lax.iota(jnp.int32, L)

    def idx_step(i, carry):
        off = i * L
        yv = y_v[pl.ds(off, L)]
        idx_v[pl.ds(off, L)] = yv * HWN + (lin + off + lane)
        return carry

    lax.fori_loop(0, NV, idx_step, 0)

    def fire(j, carry):
        o = j * CH
        pltpu.async_copy(
            scores_hbm.at[idx_v.at[pl.ds(o, CH)]],
            val_v.at[pl.ds(o, CH)],
            sem,
        )
        return carry

    lax.fori_loop(0, NCH, fire, 0)
    pltpu.make_async_copy(scores_hbm.at[pl.ds(0, PPW)], val_v, sem).wait()

    def red_step(i, acc):
        return acc - val_v[pl.ds(i * L, L)]

    acc = lax.fori_loop(0, NV, red_step, jnp.zeros((L,), jnp.float32))
    acc_v[...] = acc
    pltpu.sync_copy(acc_v, out_hbm.at[wid])


@functools.partial(
    pl.kernel,
    out_type=jax.ShapeDtypeStruct((NW, L), jnp.float32),
    mesh=plsc.VectorSubcoreMesh(core_axis_name="c", subcore_axis_name="s"),
    scratch_types=[
        pltpu.VMEM((PPW,), jnp.int32),
        pltpu.VMEM((PPW,), jnp.int32),
        pltpu.VMEM((PPW,), jnp.float32),
        pltpu.VMEM((L,), jnp.float32),
        pltpu.SemaphoreType.DMA,
    ],
)
def _gather_sum(scores_hbm, y_hbm, out_hbm, y_v, idx_v, val_v, acc_v, sem):
    _body(scores_hbm, y_hbm, out_hbm, y_v, idx_v, val_v, acc_v, sem)


def kernel(scores, y):
    scores_flat = scores.reshape(-1)
    y_flat = y.reshape(-1).astype(jnp.int32)
    partials = _gather_sum(scores_flat, y_flat)
    return jnp.sum(partials)


# speedup vs baseline: 1.4863x; 1.1771x over previous
import functools

import jax
import jax.numpy as jnp
from jax import lax
from jax.experimental import pallas as pl
from jax.experimental.pallas import tpu as pltpu
from jax.experimental.pallas import tpu_sc as plsc

B, C, H, W = 8, 96, 224, 224
HWN = H * W
NPIX = B * HWN
NC, NS, L = 2, 16, 16
NW = NC * NS
PPW = NPIX // NW
QH = 56
NHR = QH // 8
TW = 128
RW = W - TW
RG = RW // L
RPIX = QH * RW
RCH = 128
NRCH = RPIX // RCH


def _body(scores_hbm, rem_hbm, y_hbm, out_hbm, y_v, buf_v, idx_v, dst_v,
          acc_v, sem):
    c = lax.axis_index("c")
    s = lax.axis_index("s")
    wid = s * NC + c
    b = wid // 4
    h0 = (wid % 4) * QH
    base = wid * PPW

    pltpu.sync_copy(y_hbm.at[pl.ds(base, PPW)], y_v)

    lane = lax.iota(jnp.int32, L)

    def tile_band(hr, acc):
        habs = h0 + hr * 8
        pltpu.sync_copy(
            scores_hbm.at[b, :, pl.ds(habs, 8), pl.ds(0, TW)], buf_v
        )

        def grp(i, acc):
            hs = i // 8
            g = i % 8
            yv = y_v[pl.ds((hr * 8 + hs) * W + g * L, L)]
            hv = jnp.broadcast_to(hs, (L,)).astype(jnp.int32)
            wv = g * L + lane
            return acc - plsc.load_gather(buf_v, [yv, hv, wv])

        return lax.fori_loop(0, 64, grp, acc)

    acc = lax.fori_loop(0, NHR, tile_band, jnp.zeros((L,), jnp.float32))

    def idx_step(i, carry):
        h = i // RG
        g = i % RG
        yv = y_v[pl.ds(h * W + TW + g * L, L)]
        idx_v[pl.ds(i * L, L)] = yv * (C * H * RW // C) + (
            b * (C * H * RW) + (h0 + h) * RW + g * L + lane
        )
        return carry

    lax.fori_loop(0, QH * RG, idx_step, 0)

    def fire(j, carry):
        o = j * RCH
        pltpu.async_copy(
            rem_hbm.at[idx_v.at[pl.ds(o, RCH)]], dst_v.at[pl.ds(o, RCH)], sem
        )
        return carry

    lax.fori_loop(0, NRCH, fire, 0)
    pltpu.make_async_copy(rem_hbm.at[pl.ds(0, RPIX)], dst_v, sem).wait()

    def red(i, acc):
        return acc - dst_v[pl.ds(i * L, L)]

    acc = lax.fori_loop(0, QH * RG, red, acc)

    acc_v[...] = acc
    pltpu.sync_copy(acc_v, out_hbm.at[wid])


@functools.partial(
    pl.kernel,
    out_type=jax.ShapeDtypeStruct((NW, L), jnp.float32),
    mesh=plsc.VectorSubcoreMesh(core_axis_name="c", subcore_axis_name="s"),
    scratch_types=[
        pltpu.VMEM((PPW,), jnp.int32),
        pltpu.VMEM((C, 8, TW), jnp.float32),
        pltpu.VMEM((RPIX,), jnp.int32),
        pltpu.VMEM((RPIX,), jnp.float32),
        pltpu.VMEM((L,), jnp.float32),
        pltpu.SemaphoreType.DMA,
    ],
    compiler_params=pltpu.CompilerParams(
        use_tc_tiling_on_sc=True, needs_layout_passes=False
    ),
)
def _gather_sum(scores_hbm, rem_hbm, y_hbm, out_hbm, y_v, buf_v, idx_v,
                dst_v, acc_v, sem):
    _body(scores_hbm, rem_hbm, y_hbm, out_hbm, y_v, buf_v, idx_v, dst_v,
          acc_v, sem)


def kernel(scores, y):
    rem = scores[:, :, :, TW:].reshape(-1)
    y_flat = y.reshape(-1).astype(jnp.int32)
    partials = _gather_sum(scores, rem, y_flat)
    return jnp.sum(partials)


# speedup vs baseline: 1.8964x; 1.2759x over previous
import functools

import jax
import jax.numpy as jnp
from jax import lax
from jax.experimental import pallas as pl
from jax.experimental.pallas import tpu as pltpu
from jax.experimental.pallas import tpu_sc as plsc

B, C, H, W = 8, 96, 224, 224
HWN = H * W
NPIX = B * HWN
NC, NS, L = 2, 16, 16
NW = NC * NS
PPW = NPIX // NW
QH = 56
NHR = QH // 8
TW = 128
SCW = 128
SCG = SCW // L
TCW = 128
TCV = W - SCW


def _sc_body(scores_hbm, y_hbm, out_hbm, y_v, buf_v, acc_v):
    c = lax.axis_index("c")
    s = lax.axis_index("s")
    wid = s * NC + c
    b = wid // 4
    h0 = (wid % 4) * QH
    base = wid * PPW

    pltpu.sync_copy(y_hbm.at[pl.ds(base, PPW)], y_v)
    lane = lax.iota(jnp.int32, L)

    def tile_band(hr, acc):
        habs = h0 + hr * 8
        pltpu.sync_copy(
            scores_hbm.at[b, :, pl.ds(habs, 8), pl.ds(0, TW)], buf_v
        )

        def grp(i, acc):
            hs = i // SCG
            g = i % SCG
            yv = y_v[pl.ds((hr * 8 + hs) * W + g * L, L)]
            hv = jnp.broadcast_to(hs, (L,)).astype(jnp.int32)
            wv = g * L + lane
            return acc - plsc.load_gather(buf_v, [yv, hv, wv])

        return lax.fori_loop(0, 8 * SCG, grp, acc)

    acc = lax.fori_loop(0, NHR, tile_band, jnp.zeros((L,), jnp.float32))
    acc_v[...] = acc
    pltpu.sync_copy(acc_v, out_hbm.at[wid])


@functools.partial(
    pl.kernel,
    out_type=jax.ShapeDtypeStruct((NW, L), jnp.float32),
    mesh=plsc.VectorSubcoreMesh(core_axis_name="c", subcore_axis_name="s"),
    scratch_types=[
        pltpu.VMEM((PPW,), jnp.int32),
        pltpu.VMEM((C, 8, TW), jnp.float32),
        pltpu.VMEM((L,), jnp.float32),
    ],
    compiler_params=pltpu.CompilerParams(
        use_tc_tiling_on_sc=True, needs_layout_passes=False
    ),
)
def _sc_gather_sum(scores_hbm, y_hbm, out_hbm, y_v, buf_v, acc_v):
    _sc_body(scores_hbm, y_hbm, out_hbm, y_v, buf_v, acc_v)


def _tc_body(s_ref, y_ref, out_ref):
    s = s_ref[0]
    y = y_ref[0]
    ge32 = jnp.broadcast_to((y >= 32)[None], (32, 8, TCW))
    ge64 = jnp.broadcast_to((y >= 64)[None], (32, 8, TCW))
    v = jnp.where(ge64, s[64:96], jnp.where(ge32, s[32:64], s[0:32]))
    k = 16
    while k >= 1:
        bit = jnp.broadcast_to((y & k) > 0, (k, 8, TCW))
        v = jnp.where(bit, v[k:2 * k], v[:k])
        k //= 2
    valid = lax.broadcasted_iota(jnp.int32, (1, 8, TCW), 2) < TCV

    @pl.when((pl.program_id(0) == 0) & (pl.program_id(1) == 0))
    def _():
        out_ref[0, 0] = 0.0

    out_ref[0, 0] -= jnp.sum(jnp.where(valid, v, 0.0))


def _tc_masked_sum(scores, y):
    return pl.pallas_call(
        _tc_body,
        grid=(B, H // 8),
        in_specs=[
            pl.BlockSpec((1, C, 8, TCW), lambda b, h: (b, 0, h, 1)),
            pl.BlockSpec((1, 8, TCW), lambda b, h: (b, h, 1)),
        ],
        out_specs=pl.BlockSpec(
            (1, 1), lambda b, h: (0, 0), memory_space=pltpu.SMEM
        ),
        out_shape=jax.ShapeDtypeStruct((1, 1), jnp.float32),
    )(scores, y)


def kernel(scores, y):
    y_flat = y.reshape(-1).astype(jnp.int32)
    sc_partials = _sc_gather_sum(scores, y_flat)
    tc_partials = _tc_masked_sum(scores, y)
    return jnp.sum(sc_partials) + tc_partials[0, 0]


# speedup vs baseline: 3.9713x; 2.0942x over previous
import functools

import jax
import jax.numpy as jnp
from jax import lax
from jax.experimental import pallas as pl
from jax.experimental.pallas import tpu as pltpu
from jax.experimental.pallas import tpu_sc as plsc

B, C, H, W = 8, 96, 224, 224
HWN = H * W
NPIX = B * HWN
NC, NS, L = 2, 16, 16
NW = NC * NS
PPW = NPIX // NW
QH = 56
NHR = QH // 8
TW = 128
SCW = 128
SCG = SCW // L
TCW = 128
TCV = W - SCW


def _sc_body(scores_hbm, y_hbm, out_hbm, y_v, buf_v, acc_v):
    c = lax.axis_index("c")
    s = lax.axis_index("s")
    wid = s * NC + c
    b = wid // 4
    h0 = (wid % 4) * QH
    base = wid * PPW

    pltpu.sync_copy(y_hbm.at[pl.ds(base, PPW)], y_v)
    lane = lax.iota(jnp.int32, L)

    def tile_band(hr, acc):
        habs = h0 + hr * 8
        pltpu.sync_copy(
            scores_hbm.at[b, :, pl.ds(habs, 8), pl.ds(0, TW)], buf_v
        )

        def grp(i, acc):
            hs = i // SCG
            g = i % SCG
            yv = y_v[pl.ds((hr * 8 + hs) * W + g * L, L)]
            hv = jnp.broadcast_to(hs, (L,)).astype(jnp.int32)
            wv = g * L + lane
            return acc - plsc.load_gather(buf_v, [yv, hv, wv])

        return lax.fori_loop(0, 8 * SCG, grp, acc)

    acc = lax.fori_loop(0, NHR, tile_band, jnp.zeros((L,), jnp.float32))
    acc_v[...] = acc
    pltpu.sync_copy(acc_v, out_hbm.at[wid])


@functools.partial(
    pl.kernel,
    out_type=jax.ShapeDtypeStruct((NW, L), jnp.float32),
    mesh=plsc.VectorSubcoreMesh(core_axis_name="c", subcore_axis_name="s"),
    scratch_types=[
        pltpu.VMEM((PPW,), jnp.int32),
        pltpu.VMEM((C, 8, TW), jnp.float32),
        pltpu.VMEM((L,), jnp.float32),
    ],
    compiler_params=pltpu.CompilerParams(
        use_tc_tiling_on_sc=True, needs_layout_passes=False
    ),
)
def _sc_gather_sum(scores_hbm, y_hbm, out_hbm, y_v, buf_v, acc_v):
    _sc_body(scores_hbm, y_hbm, out_hbm, y_v, buf_v, acc_v)


TCH = 56


def _tc_body(s_ref, y_ref, out_ref):
    s = s_ref[0]
    y = y_ref[0]
    ge32 = jnp.broadcast_to((y >= 32)[None], (32, TCH, TCW))
    ge64 = jnp.broadcast_to((y >= 64)[None], (32, TCH, TCW))
    v = jnp.where(ge64, s[64:96], jnp.where(ge32, s[32:64], s[0:32]))
    k = 16
    while k >= 1:
        bit = jnp.broadcast_to((y & k) > 0, (k, TCH, TCW))
        v = jnp.where(bit, v[k:2 * k], v[:k])
        k //= 2
    valid = lax.broadcasted_iota(jnp.int32, (1, TCH, TCW), 2) < TCV

    @pl.when((pl.program_id(0) == 0) & (pl.program_id(1) == 0))
    def _():
        out_ref[0, 0] = 0.0

    out_ref[0, 0] -= jnp.sum(jnp.where(valid, v, 0.0))


def _tc_masked_sum(scores, y):
    return pl.pallas_call(
        _tc_body,
        grid=(B, H // TCH),
        in_specs=[
            pl.BlockSpec((1, C, TCH, TCW), lambda b, h: (b, 0, h, 1)),
            pl.BlockSpec((1, TCH, TCW), lambda b, h: (b, h, 1)),
        ],
        out_specs=pl.BlockSpec(
            (1, 1), lambda b, h: (0, 0), memory_space=pltpu.SMEM
        ),
        out_shape=jax.ShapeDtypeStruct((1, 1), jnp.float32),
    )(scores, y)


def kernel(scores, y):
    y_flat = y.reshape(-1).astype(jnp.int32)
    sc_partials = _sc_gather_sum(scores, y_flat)
    tc_partials = _tc_masked_sum(scores, y)
    return jnp.sum(sc_partials) + tc_partials[0, 0]


# speedup vs baseline: 4.0696x; 1.0248x over previous
import functools

import jax
import jax.numpy as jnp
from jax import lax
from jax.experimental import pallas as pl
from jax.experimental.pallas import tpu as pltpu
from jax.experimental.pallas import tpu_sc as plsc

B, C, H, W = 8, 96, 224, 224
HWN = H * W
NPIX = B * HWN
NC, NS, L = 2, 16, 16
NW = NC * NS
PPW = NPIX // NW
QH = 56
NHR = QH // 8
TW = 128
SCW = 128
SCG = SCW // L
TCW = 128
TCV = W - SCW


def _sc_body(scores_hbm, y_hbm, out_hbm, y_v, buf_v, acc_v):
    c = lax.axis_index("c")
    s = lax.axis_index("s")
    wid = s * NC + c
    b = wid // 4
    h0 = (wid % 4) * QH

    pltpu.sync_copy(
        y_hbm.at[b, pl.ds(h0, QH), pl.ds(0, SCW)], y_v
    )
    lane = lax.iota(jnp.int32, L)

    def tile_band(hr, acc):
        habs = h0 + hr * 8
        pltpu.sync_copy(
            scores_hbm.at[b, :, pl.ds(habs, 8), pl.ds(0, TW)], buf_v
        )

        def grp(i, acc):
            hs = i // SCG
            g = i % SCG
            yv = y_v[hr * 8 + hs, pl.ds(g * L, L)]
            hv = jnp.broadcast_to(hs, (L,)).astype(jnp.int32)
            wv = g * L + lane
            return acc - plsc.load_gather(buf_v, [yv, hv, wv])

        return lax.fori_loop(0, 8 * SCG, grp, acc)

    acc = lax.fori_loop(0, NHR, tile_band, jnp.zeros((L,), jnp.float32))
    acc_v[...] = acc
    pltpu.sync_copy(acc_v, out_hbm.at[wid])


@functools.partial(
    pl.kernel,
    out_type=jax.ShapeDtypeStruct((NW, L), jnp.float32),
    mesh=plsc.VectorSubcoreMesh(core_axis_name="c", subcore_axis_name="s"),
    scratch_types=[
        pltpu.VMEM((QH, SCW), jnp.int32),
        pltpu.VMEM((C, 8, TW), jnp.float32),
        pltpu.VMEM((L,), jnp.float32),
    ],
    compiler_params=pltpu.CompilerParams(
        use_tc_tiling_on_sc=True, needs_layout_passes=False
    ),
)
def _sc_gather_sum(scores_hbm, y_hbm, out_hbm, y_v, buf_v, acc_v):
    _sc_body(scores_hbm, y_hbm, out_hbm, y_v, buf_v, acc_v)


TCH = 56


def _tc_body(s_ref, y_ref, out_ref):
    s = s_ref[0]
    y = y_ref[0]
    ge32 = jnp.broadcast_to((y >= 32)[None], (32, TCH, TCW))
    ge64 = jnp.broadcast_to((y >= 64)[None], (32, TCH, TCW))
    v = jnp.where(ge64, s[64:96], jnp.where(ge32, s[32:64], s[0:32]))
    k = 16
    while k >= 1:
        bit = jnp.broadcast_to((y & k) > 0, (k, TCH, TCW))
        v = jnp.where(bit, v[k:2 * k], v[:k])
        k //= 2
    valid = lax.broadcasted_iota(jnp.int32, (1, TCH, TCW), 2) < TCV

    @pl.when((pl.program_id(0) == 0) & (pl.program_id(1) == 0))
    def _():
        out_ref[0, 0] = 0.0

    out_ref[0, 0] -= jnp.sum(jnp.where(valid, v, 0.0))


def _tc_masked_sum(scores, y):
    return pl.pallas_call(
        _tc_body,
        grid=(B, H // TCH),
        in_specs=[
            pl.BlockSpec((1, C, TCH, TCW), lambda b, h: (b, 0, h, 1)),
            pl.BlockSpec((1, TCH, TCW), lambda b, h: (b, h, 1)),
        ],
        out_specs=pl.BlockSpec(
            (1, 1), lambda b, h: (0, 0), memory_space=pltpu.SMEM
        ),
        out_shape=jax.ShapeDtypeStruct((1, 1), jnp.float32),
    )(scores, y)


def kernel(scores, y):
    sc_partials = _sc_gather_sum(scores, y)
    tc_partials = _tc_masked_sum(scores, y)
    return jnp.sum(sc_partials) + tc_partials[0, 0]


# speedup vs baseline: 4.1347x; 1.0160x over previous
import functools

import jax
import jax.numpy as jnp
from jax import lax
from jax.experimental import pallas as pl
from jax.experimental.pallas import tpu as pltpu
from jax.experimental.pallas import tpu_sc as plsc

B, C, H, W = 8, 96, 224, 224
HWN = H * W
NPIX = B * HWN
NC, NS, L = 2, 16, 16
NW = NC * NS
PPW = NPIX // NW
QH = 56
NHR = QH // 8
TW = 128
SCW = 128
SCG = SCW // L
TCW = 128
TCV = W - SCW


def _sc_body(scores_hbm, y_hbm, out_hbm, y_v, buf_v, acc_v):
    c = lax.axis_index("c")
    s = lax.axis_index("s")
    wid = s * NC + c
    b = wid // 4
    h0 = (wid % 4) * QH

    pltpu.sync_copy(
        y_hbm.at[b, pl.ds(h0, QH), pl.ds(0, SCW)], y_v
    )
    lane = lax.iota(jnp.int32, L)

    def tile_band(hr, acc):
        habs = h0 + hr * 8
        pltpu.sync_copy(
            scores_hbm.at[b, :, pl.ds(habs, 8), pl.ds(0, TW)], buf_v
        )

        def grp(i, acc):
            hs = i // SCG
            g = i % SCG
            yv = y_v[hr * 8 + hs, pl.ds(g * L, L)]
            hv = jnp.broadcast_to(hs, (L,)).astype(jnp.int32)
            wv = g * L + lane
            return acc - plsc.load_gather(buf_v, [yv, hv, wv])

        return lax.fori_loop(0, 8 * SCG, grp, acc)

    acc = lax.fori_loop(0, NHR, tile_band, jnp.zeros((L,), jnp.float32))
    acc_v[...] = acc
    pltpu.sync_copy(acc_v, out_hbm.at[wid])


@functools.partial(
    pl.kernel,
    out_type=jax.ShapeDtypeStruct((NW, L), jnp.float32),
    mesh=plsc.VectorSubcoreMesh(core_axis_name="c", subcore_axis_name="s"),
    scratch_types=[
        pltpu.VMEM((QH, SCW), jnp.int32),
        pltpu.VMEM((C, 8, TW), jnp.float32),
        pltpu.VMEM((L,), jnp.float32),
    ],
    compiler_params=pltpu.CompilerParams(
        use_tc_tiling_on_sc=True, needs_layout_passes=False
    ),
)
def _sc_gather_sum(scores_hbm, y_hbm, out_hbm, y_v, buf_v, acc_v):
    _sc_body(scores_hbm, y_hbm, out_hbm, y_v, buf_v, acc_v)


TCH = 112


def _tc_body(s_ref, y_ref, out_ref):
    s = s_ref[0]
    y = y_ref[0]
    ge32 = jnp.broadcast_to((y >= 32)[None], (32, TCH, TCW))
    ge64 = jnp.broadcast_to((y >= 64)[None], (32, TCH, TCW))
    v = jnp.where(ge64, s[64:96], jnp.where(ge32, s[32:64], s[0:32]))
    k = 16
    while k >= 1:
        bit = jnp.broadcast_to((y & k) > 0, (k, TCH, TCW))
        v = jnp.where(bit, v[k:2 * k], v[:k])
        k //= 2
    valid = lax.broadcasted_iota(jnp.int32, (1, TCH, TCW), 2) < TCV

    @pl.when((pl.program_id(0) == 0) & (pl.program_id(1) == 0))
    def _():
        out_ref[0, 0] = 0.0

    out_ref[0, 0] -= jnp.sum(jnp.where(valid, v, 0.0))


def _tc_masked_sum(scores, y):
    return pl.pallas_call(
        _tc_body,
        grid=(B, H // TCH),
        in_specs=[
            pl.BlockSpec((1, C, TCH, TCW), lambda b, h: (b, 0, h, 1)),
            pl.BlockSpec((1, TCH, TCW), lambda b, h: (b, h, 1)),
        ],
        out_specs=pl.BlockSpec(
            (1, 1), lambda b, h: (0, 0), memory_space=pltpu.SMEM
        ),
        out_shape=jax.ShapeDtypeStruct((1, 1), jnp.float32),
    )(scores, y)


def kernel(scores, y):
    sc_partials = _sc_gather_sum(scores, y)
    tc_partials = _tc_masked_sum(scores, y)
    return jnp.sum(sc_partials) + tc_partials[0, 0]


# speedup vs baseline: 4.1706x; 1.0087x over previous
import functools

import jax
import jax.numpy as jnp
from jax import lax
from jax.experimental import pallas as pl
from jax.experimental.pallas import tpu as pltpu
from jax.experimental.pallas import tpu_sc as plsc

B, C, H, W = 8, 96, 224, 224
HWN = H * W
NPIX = B * HWN
NC, NS, L = 2, 16, 16
NW = NC * NS
PPW = NPIX // NW
QH = 56
NHR = QH // 8
TW = 128
SCW = 128
SCG = SCW // L
TCW = 128
TCV = W - SCW


def _sc_body(scores_hbm, y_hbm, out_hbm, y_v, buf_v, acc_v):
    c = lax.axis_index("c")
    s = lax.axis_index("s")
    wid = s * NC + c
    b = wid // 4
    h0 = (wid % 4) * QH

    pltpu.sync_copy(
        y_hbm.at[b, pl.ds(h0, QH), pl.ds(0, SCW)], y_v
    )
    lane = lax.iota(jnp.int32, L)

    def tile_band(hr, acc):
        habs = h0 + hr * 8
        pltpu.sync_copy(
            scores_hbm.at[b, :, pl.ds(habs, 8), pl.ds(0, TW)], buf_v
        )

        def grp(i, acc):
            hs = i // SCG
            g = i % SCG
            yv = y_v[hr * 8 + hs, pl.ds(g * L, L)]
            hv = jnp.broadcast_to(hs, (L,)).astype(jnp.int32)
            wv = g * L + lane
            return acc - plsc.load_gather(buf_v, [yv, hv, wv])

        return lax.fori_loop(0, 8 * SCG, grp, acc)

    acc = lax.fori_loop(0, NHR, tile_band, jnp.zeros((L,), jnp.float32))
    acc_v[...] = acc
    pltpu.sync_copy(acc_v, out_hbm.at[wid])


@functools.partial(
    pl.kernel,
    out_type=jax.ShapeDtypeStruct((NW, L), jnp.float32),
    mesh=plsc.VectorSubcoreMesh(core_axis_name="c", subcore_axis_name="s"),
    scratch_types=[
        pltpu.VMEM((QH, SCW), jnp.int32),
        pltpu.VMEM((C, 8, TW), jnp.float32),
        pltpu.VMEM((L,), jnp.float32),
    ],
    compiler_params=pltpu.CompilerParams(
        use_tc_tiling_on_sc=True, needs_layout_passes=False
    ),
)
def _sc_gather_sum(scores_hbm, y_hbm, out_hbm, y_v, buf_v, acc_v):
    _sc_body(scores_hbm, y_hbm, out_hbm, y_v, buf_v, acc_v)


TCH = 224


def _tc_body(s_ref, y_ref, out_ref):
    s = s_ref[0]
    y = y_ref[0]
    ge32 = jnp.broadcast_to((y >= 32)[None], (32, TCH, TCW))
    ge64 = jnp.broadcast_to((y >= 64)[None], (32, TCH, TCW))
    v = jnp.where(ge64, s[64:96], jnp.where(ge32, s[32:64], s[0:32]))
    k = 16
    while k >= 1:
        bit = jnp.broadcast_to((y & k) > 0, (k, TCH, TCW))
        v = jnp.where(bit, v[k:2 * k], v[:k])
        k //= 2
    valid = lax.broadcasted_iota(jnp.int32, (1, TCH, TCW), 2) < TCV

    @pl.when((pl.program_id(0) == 0) & (pl.program_id(1) == 0))
    def _():
        out_ref[0, 0] = 0.0

    out_ref[0, 0] -= jnp.sum(jnp.where(valid, v, 0.0))


def _tc_masked_sum(scores, y):
    return pl.pallas_call(
        _tc_body,
        grid=(B, H // TCH),
        in_specs=[
            pl.BlockSpec((1, C, TCH, TCW), lambda b, h: (b, 0, h, 1)),
            pl.BlockSpec((1, TCH, TCW), lambda b, h: (b, h, 1)),
        ],
        out_specs=pl.BlockSpec(
            (1, 1), lambda b, h: (0, 0), memory_space=pltpu.SMEM
        ),
        out_shape=jax.ShapeDtypeStruct((1, 1), jnp.float32),
    )(scores, y)


def kernel(scores, y):
    sc_partials = _sc_gather_sum(scores, y)
    tc_partials = _tc_masked_sum(scores, y)
    return jnp.sum(sc_partials) + tc_partials[0, 0]
